# Initial kernel scaffold; baseline (speedup 1.0000x reference)
import jax
import jax.numpy as jnp
from jax.experimental import pallas as pl


def kernel(feat, biases, all_h1s, all_w1s, all_h2s, all_w2s):
    raise NotImplementedError("write your pallas kernel here")


# speedup vs baseline: 22.5077x; 22.5077x over previous
import functools

import jax
import jax.numpy as jnp
from jax import lax
from jax.experimental import pallas as pl
from jax.experimental.pallas import tpu as pltpu, tpu_sc as plsc

_R = 8
_K = 2 * _R + 1

_NC = 2
_NS = 16
_NW = _NC * _NS
_NPH = 8


@functools.lru_cache(maxsize=None)
def _build_fill(H: int, W: int):
    TH = 2 * H - 1
    TWP = ((2 * W - 1) + 15) // 16 * 16
    r0 = H - 1 - _R
    c0 = W - 1 - _R
    tiles = H * W
    per = tiles // _NW
    CH = 8
    nzc = TWP // 16

    mesh = plsc.VectorSubcoreMesh(
        core_axis_name="c", subcore_axis_name="s",
        num_cores=_NC, num_subcores=_NS)

    @functools.partial(
        pl.kernel,
        out_type=jax.ShapeDtypeStruct((H, W, H, W), jnp.float32),
        mesh=mesh,
        scratch_types=[
            pltpu.VMEM((_K, _K), jnp.float32),
            pltpu.VMEM((TH, TWP), jnp.float32),
            pltpu.VMEM_SHARED((_NPH, TH, TWP), jnp.float32),
            pltpu.SemaphoreType.DMA,
        ],
        compiler_params=pltpu.CompilerParams(use_tc_tiling_on_sc=False),
    )
    def fill(biases_hbm, out_hbm, bv, tbuf, phases, sem):
        s = lax.axis_index("s")

        @pl.when(s < _NPH)
        def _build():
            pltpu.sync_copy(biases_hbm, bv)

            def zbody(r, carry):
                for j in range(nzc):
                    tbuf[r, pl.ds(j * 16, 16)] = jnp.zeros((16,), jnp.float32)
                return carry

            lax.fori_loop(0, TH, zbody, 0)
            for r in range(_K):
                tbuf[r0 + r, pl.ds(c0 - s, 16)] = bv[r, pl.ds(0, 16)]
                tbuf[r0 + r, pl.ds(c0 - s + 1, 16)] = bv[r, pl.ds(1, 16)]
            pltpu.sync_copy(tbuf, phases.at[s])

        plsc.subcore_barrier()

        wid = s * _NC + lax.axis_index("c")
        base = wid * per

        def obody(j, carry):
            p0 = base + j * CH
            cps = []
            for b in range(CH):
                p = p0 + b
                h1 = p // W
                w1 = p - h1 * W
                c = (W - 1) - w1
                ph = lax.rem(c, _NPH)
                cq = pl.multiple_of(c - ph, _NPH)
                cps.append(pltpu.async_copy(
                    phases.at[ph, pl.ds(H - 1 - h1, H), pl.ds(cq, W)],
                    out_hbm.at[h1, w1],
                    sem))
            for cp in cps:
                cp.wait()
            return carry

        lax.fori_loop(0, per // CH, obody, 0)

    return fill


def kernel(feat, biases, all_h1s, all_w1s, all_h2s, all_w2s):
    H, W = feat.shape[-2], feat.shape[-1]
    out = _build_fill(H, W)(biases.astype(jnp.float32))
    out = out.astype(feat.dtype)
    for _ in range(feat.ndim - 2):
        out = out[None]
    return out
